# baseline (speedup 1.0000x reference)
import functools

import jax
import jax.numpy as jnp
from jax import lax
from jax.experimental import pallas as pl
from jax.experimental.pallas import tpu as pltpu
from jax.experimental.pallas import tpu_sc as plsc

D_MODEL_ = 1024
N_ROWS_ = 32768
NUM_CORES_ = 2
NUM_SUBCORES_ = 16
NUM_WORKERS_ = NUM_CORES_ * NUM_SUBCORES_
ROWS_PER_W_ = N_ROWS_ // NUM_WORKERS_
CHUNK_ = 32
NCHUNK_ = ROWS_PER_W_ // CHUNK_
LANES_ = 16
COLS_PER_PASS_ = 256
NPASS_ = D_MODEL_ // COLS_PER_PASS_
KVEC_ = COLS_PER_PASS_ // LANES_


def _make_sc_embed():
    mesh = plsc.VectorSubcoreMesh(core_axis_name="c", subcore_axis_name="s")

    @functools.partial(
        pl.kernel,
        out_type=jax.ShapeDtypeStruct((N_ROWS_, D_MODEL_), jnp.float32),
        mesh=mesh,
        scratch_types=[
            pltpu.VMEM((ROWS_PER_W_,), jnp.int32),
            pltpu.VMEM((2, D_MODEL_), jnp.float32),
            pltpu.VMEM((CHUNK_, D_MODEL_), jnp.float32),
            pltpu.VMEM((CHUNK_, D_MODEL_), jnp.float32),
            pltpu.SemaphoreType.DMA,
            pltpu.SemaphoreType.DMA,
        ],
    )
    def sc_embed(idx_hbm, table_hbm, out_hbm, idx_v, wv, buf0, buf1, s0, s1):
        cid = lax.axis_index("c")
        sid = lax.axis_index("s")
        wid = sid * NUM_CORES_ + cid
        base = wid * ROWS_PER_W_

        pltpu.sync_copy(table_hbm, wv)
        pltpu.sync_copy(idx_hbm.at[pl.ds(base, ROWS_PER_W_)], idx_v)

        bufs = (buf0, buf1)
        sems = (s0, s1)

        def compute_chunk(c, buf):
            cbase = c * CHUNK_
            for jp in range(NPASS_):
                col0 = jp * COLS_PER_PASS_
                w0s = [wv[0, pl.ds(col0 + k * LANES_, LANES_)]
                       for k in range(KVEC_)]
                w1s = [wv[1, pl.ds(col0 + k * LANES_, LANES_)]
                       for k in range(KVEC_)]
                dfs = [w1s[k] - w0s[k] for k in range(KVEC_)]
                for h in range(CHUNK_ // LANES_):
                    th = idx_v[pl.ds(cbase + h * LANES_, LANES_)].astype(
                        jnp.float32)

                    def row_body(r2, _, th=th, h=h, w0s=w0s, dfs=dfs,
                                 col0=col0):
                        for u in range(2):
                            r = 2 * r2 + u
                            t = lax.gather(
                                th,
                                jnp.full((LANES_, 1), r, jnp.int32),
                                lax.GatherDimensionNumbers(
                                    offset_dims=(),
                                    collapsed_slice_dims=(0,),
                                    start_index_map=(0,)),
                                slice_sizes=(1,),
                                mode=lax.GatherScatterMode.PROMISE_IN_BOUNDS)
                            row = h * LANES_ + r
                            for k in range(KVEC_):
                                buf[row, pl.ds(col0 + k * LANES_, LANES_)] = (
                                    w0s[k] + t * dfs[k])
                        return 0

                    lax.fori_loop(0, LANES_ // 2, row_body, 0)

        def start_write(c, bi):
            return pltpu.async_copy(
                bufs[bi],
                out_hbm.at[pl.ds(base + c * CHUNK_, CHUNK_)],
                sems[bi])

        def wait_write(bi):
            pltpu.make_async_copy(
                bufs[bi], out_hbm.at[pl.ds(base, CHUNK_)], sems[bi]).wait()

        compute_chunk(0, buf0)
        start_write(0, 0)
        compute_chunk(1, buf1)
        start_write(1, 1)

        def pair_body(m, _):
            for bi in range(2):
                c = 2 * m + bi
                wait_write(bi)
                compute_chunk(c, bufs[bi])
                start_write(c, bi)
            return 0

        lax.fori_loop(1, NCHUNK_ // 2, pair_body, 0)
        wait_write(0)
        wait_write(1)

    return sc_embed


_sc_embed = _make_sc_embed()


def kernel(token_types, table):
    b, s = token_types.shape
    idx = token_types.reshape(b * s).astype(jnp.int32)
    out = _sc_embed(idx, table)
    return out.reshape(b, s, D_MODEL_)


# speedup vs baseline: 1.1090x; 1.1090x over previous
import functools

import jax
import jax.numpy as jnp
from jax import lax
from jax.experimental import pallas as pl
from jax.experimental.pallas import tpu as pltpu
from jax.experimental.pallas import tpu_sc as plsc

D_MODEL_ = 1024
N_ROWS_ = 32768
NUM_CORES_ = 2
NUM_SUBCORES_ = 16
NUM_WORKERS_ = NUM_CORES_ * NUM_SUBCORES_
ROWS_PER_W_ = N_ROWS_ // NUM_WORKERS_
LANES_ = 16
NGROUPS_ = ROWS_PER_W_ // LANES_


def _make_sc_embed():
    mesh = plsc.VectorSubcoreMesh(core_axis_name="c", subcore_axis_name="s")

    @functools.partial(
        pl.kernel,
        out_type=jax.ShapeDtypeStruct((N_ROWS_, D_MODEL_), jnp.float32),
        mesh=mesh,
        scratch_types=[
            pltpu.VMEM((ROWS_PER_W_,), jnp.int32),
            pltpu.VMEM((2, D_MODEL_), jnp.float32),
            pltpu.SemaphoreType.DMA,
        ],
    )
    def sc_embed(idx_hbm, table_hbm, out_hbm, idx_v, wv, sem):
        cid = lax.axis_index("c")
        sid = lax.axis_index("s")
        wid = sid * NUM_CORES_ + cid
        base = wid * ROWS_PER_W_

        pltpu.sync_copy(table_hbm, wv)
        pltpu.sync_copy(idx_hbm.at[pl.ds(base, ROWS_PER_W_)], idx_v)

        def group_body(g, _):
            th = idx_v[pl.ds(g * LANES_, LANES_)]
            row0 = base + g * LANES_
            for u in range(LANES_):
                t = jnp.clip(th[u], 0, 1)
                pltpu.async_copy(wv.at[t], out_hbm.at[row0 + u], sem)
            return 0

        lax.fori_loop(0, NGROUPS_, group_body, 0)

        pltpu.make_async_copy(
            out_hbm.at[pl.ds(0, ROWS_PER_W_)],
            out_hbm.at[pl.ds(base, ROWS_PER_W_)],
            sem).wait()

    return sc_embed


_sc_embed = _make_sc_embed()


def kernel(token_types, table):
    b, s = token_types.shape
    idx = token_types.reshape(b * s).astype(jnp.int32)
    out = _sc_embed(idx, table)
    return out.reshape(b, s, D_MODEL_)
